# baseline (speedup 1.0000x reference)
import functools

import jax
import jax.numpy as jnp
from jax import lax
from jax.experimental import pallas as pl
from jax.experimental.pallas import tpu as pltpu
from jax.experimental.pallas import tpu_sc as plsc

_FIELD_DIM = 38462
_NUM_FIELDS = 26
_EMBED_DIM = 16
_ROW = 2 * _EMBED_DIM
_TOTAL = _FIELD_DIM * _NUM_FIELDS

_BATCH = 4096
_NW = 32
_BPW = _BATCH // _NW


def _fm_body(xt_hbm, emb_hbm, lin_hbm, bias_hbm, out_hbm,
             idx_v, lin_v, buf, rbuf, obuf, bias_v, sem_lin, sem):
    c = lax.axis_index("c")
    s = lax.axis_index("s")
    w = s * 2 + c

    with jax.named_scope("idx_setup"):
        pltpu.sync_copy(xt_hbm.at[:, pl.ds(w * _BPW, _BPW)], idx_v)
        pltpu.sync_copy(bias_hbm, bias_v)

        for j in range(_NUM_FIELDS):
            off = jnp.int32(j * _FIELD_DIM)
            for k in range(_BPW // 16):
                idx_v[j, pl.ds(k * 16, 16)] = idx_v[j, pl.ds(k * 16, 16)] + off

    with jax.named_scope("gather_start"):
        emb_descs = [
            pltpu.async_copy(emb_hbm.at[idx_v.at[j]], buf.at[j], sem)
            for j in range(_NUM_FIELDS)
        ]
        lin_descs = [
            pltpu.async_copy(lin_hbm.at[idx_v.at[j]], lin_v.at[j], sem_lin)
            for j in range(_NUM_FIELDS)
        ]

    with jax.named_scope("gather_wait"):
        for q in emb_descs:
            q.wait()

    zeros = jnp.zeros((16,), jnp.float32)
    lanes = lax.iota(jnp.int32, 16)

    with jax.named_scope("powersums"):
        def sbody(i, carry):
            s1lo = zeros
            s2lo = zeros
            s1 = zeros
            s2 = zeros
            s3 = zeros
            for j in range(_NUM_FIELDS):
                vlo = buf[j, i, pl.ds(0, 16)]
                vhi = buf[j, i, pl.ds(16, 16)]
                s1lo = s1lo + vlo
                s2lo = s2lo + vlo * vlo
                q2 = vhi * vhi
                s1 = s1 + vhi
                s2 = s2 + q2
                s3 = s3 + q2 * vhi
            e2 = 0.5 * (s1lo * s1lo - s2lo)
            e3 = (s1 * s1 * s1 - 3.0 * s1 * s2 + 2.0 * s3) * (1.0 / 6.0)
            rbuf[pl.ds(i * 16, 16)] = e2 + e3
            return carry

        lax.fori_loop(0, _BPW, sbody, 0)

    with jax.named_scope("lin_wait"):
        for q in lin_descs:
            q.wait()

    with jax.named_scope("finish"):
        for ch in range(_BPW // 16):
            acc = zeros
            for d in range(16):
                acc = acc + plsc.load_gather(
                    rbuf, [lanes * 16 + jnp.int32(ch * 256 + d)])
            for j in range(_NUM_FIELDS):
                acc = acc + lin_v[j, pl.ds(ch * 16, 16)]
            y = acc + bias_v[...]
            obuf[pl.ds(ch * 16, 16)] = 1.0 / (1.0 + jnp.exp(-y))

        pltpu.sync_copy(obuf, out_hbm.at[pl.ds(w * _BPW, _BPW)])


@jax.jit
def _fm_sc(xt, emb, lin1d, bias16):
    mesh = plsc.VectorSubcoreMesh(core_axis_name="c", subcore_axis_name="s")
    f = functools.partial(
        pl.kernel,
        mesh=mesh,
        out_type=jax.ShapeDtypeStruct((_BATCH,), jnp.float32),
        scratch_types=[
            pltpu.VMEM((_NUM_FIELDS, _BPW), jnp.int32),
            pltpu.VMEM((_NUM_FIELDS, _BPW), jnp.float32),
            pltpu.VMEM((_NUM_FIELDS, _BPW, _ROW), jnp.float32),
            pltpu.VMEM((_BPW * 16,), jnp.float32),
            pltpu.VMEM((_BPW,), jnp.float32),
            pltpu.VMEM((16,), jnp.float32),
            pltpu.SemaphoreType.DMA,
            pltpu.SemaphoreType.DMA,
        ],
        compiler_params=pltpu.CompilerParams(
            needs_layout_passes=False, use_tc_tiling_on_sc=False),
    )(_fm_body)
    return f(xt, emb, lin1d, bias16)


def kernel(x, emb_table, lin_table, bias):
    xt = x.astype(jnp.int32).T
    lin1d = lin_table.reshape(-1)
    bias16 = jnp.broadcast_to(bias.astype(jnp.float32), (16,))
    return _fm_sc(xt, emb_table, lin1d, bias16)
